# Initial kernel scaffold; baseline (speedup 1.0000x reference)
import jax
import jax.numpy as jnp
from jax.experimental import pallas as pl


def kernel(x, edge_index, edge_in, in_w, edge_out, out_w, lin1_w, bias1, linx0_w, biasx0, conv_w, conv_b):
    raise NotImplementedError("write your pallas kernel here")


# speedup vs baseline: 7.7144x; 7.7144x over previous
import functools
import math

import jax
import jax.numpy as jnp
from jax import lax
from jax.experimental import pallas as pl
from jax.experimental.pallas import tpu as pltpu
from jax.experimental.pallas import tpu_sc as plsc

F32 = jnp.float32

NC = 2
NS = 16
NW = NC * NS
LANES = 16

CHUNK = 128
F = 64
BN = 512


def _cdiv(a, b):
    return (a + b - 1) // b


def _make_deg_kernel(n_pad, chunks):
    rows_per_tile = n_pad // NS
    zsteps = rows_per_tile // CHUNK
    mesh = plsc.VectorSubcoreMesh(core_axis_name="c", subcore_axis_name="s")

    @functools.partial(
        pl.kernel,
        out_type=jax.ShapeDtypeStruct((NC * 3 * n_pad,), F32),
        mesh=mesh,
        scratch_types=[
            pltpu.VMEM((chunks, CHUNK), jnp.int32),
            pltpu.VMEM((chunks, CHUNK), F32),
            pltpu.VMEM((rows_per_tile,), F32),
            pltpu.VMEM_SHARED((n_pad,), F32),
            pltpu.VMEM_SHARED((n_pad,), F32),
            pltpu.VMEM_SHARED((n_pad,), F32),
        ],
    )
    def deg_kernel(src_hbm, ew_hbm, zrow_hbm, out_hbm,
                   src_v, ew_v, stage_v, acc0, acc1, acc2):
        c = lax.axis_index("c")
        s = lax.axis_index("s")
        wid = c * NS + s
        accs = [acc0, acc1, acc2]
        pltpu.sync_copy(zrow_hbm, stage_v)
        for acc in accs:
            pltpu.sync_copy(stage_v, acc.at[pl.ds(s * rows_per_tile, rows_per_tile)])
        plsc.subcore_barrier()
        for k, acc in enumerate(accs):
            pltpu.sync_copy(src_hbm.at[k, wid], src_v)
            pltpu.sync_copy(ew_hbm.at[k, wid], ew_v)

            def body(j, _, acc=acc):
                pltpu.sync_copy(ew_v.at[j], acc.at[src_v.at[j]], add=True)
                return 0

            lax.fori_loop(0, chunks, body, 0)
        plsc.subcore_barrier()
        for k, acc in enumerate(accs):
            pltpu.sync_copy(acc.at[pl.ds(s * rows_per_tile, rows_per_tile)], stage_v)
            off = (c * 3 + k) * n_pad + s * rows_per_tile
            pltpu.sync_copy(stage_v, out_hbm.at[pl.ds(off, rows_per_tile)])

    return deg_kernel


def _make_conv_kernel(n_pad, chunks, weighted):
    rows_per_tile = n_pad // NS
    zsteps = rows_per_tile // CHUNK
    mesh = plsc.VectorSubcoreMesh(core_axis_name="c", subcore_axis_name="s")

    @functools.partial(
        pl.kernel,
        out_type=jax.ShapeDtypeStruct((NC, n_pad, 2 * F), F32),
        mesh=mesh,
        scratch_types=[
            pltpu.VMEM((chunks, CHUNK), jnp.int32),
            pltpu.VMEM((chunks, CHUNK), jnp.int32),
            pltpu.VMEM((CHUNK * LANES,), F32),
            pltpu.VMEM((CHUNK, 2 * F), F32),
            pltpu.VMEM_SHARED((n_pad, 2 * F), F32),
        ],
    )
    def conv_kernel(h_hbm, src_hbm, dst_hbm, ew_hbm, zblk_hbm, out_hbm,
                    src_v, dst_v, ew_v, rows_v, acc):
        c = lax.axis_index("c")
        s = lax.axis_index("s")
        wid = c * NS + s
        pltpu.sync_copy(zblk_hbm, rows_v)
        for z in range(zsteps):
            off = s * rows_per_tile + z * CHUNK
            pltpu.sync_copy(rows_v, acc.at[pl.ds(off, CHUNK)])
        plsc.subcore_barrier()
        pltpu.sync_copy(src_hbm.at[wid], src_v)
        pltpu.sync_copy(dst_hbm.at[wid], dst_v)

        def chunk_body(j, _):
            pltpu.sync_copy(h_hbm.at[src_v.at[j]], rows_v)
            if weighted:
                off = (wid * chunks + j) * (CHUNK * LANES)
                pltpu.sync_copy(ew_hbm.at[pl.ds(off, CHUNK * LANES)], ew_v)
                for i in range(CHUNK):
                    b = ew_v[pl.ds(i * LANES, LANES)]
                    for q in range(F // LANES):
                        rows_v[i, pl.ds(q * LANES, LANES)] = (
                            rows_v[i, pl.ds(q * LANES, LANES)] * b)
            pltpu.sync_copy(rows_v, acc.at[dst_v.at[j]], add=True)
            return 0

        lax.fori_loop(0, chunks, chunk_body, 0)
        plsc.subcore_barrier()
        for z in range(zsteps):
            off = s * rows_per_tile + z * CHUNK
            pltpu.sync_copy(acc.at[pl.ds(off, CHUNK)], rows_v)
            pltpu.sync_copy(rows_v, out_hbm.at[c, pl.ds(off, CHUNK)])

    return conv_kernel


def _dis_from_deg(deg_blk):
    d = deg_blk[0] + deg_blk[1]
    return jnp.where(d > 0, lax.rsqrt(d), jnp.zeros_like(d))


def _tc1_body(x_ref, w_ref, deg_ref, h0_ref, h1_ref, h2_ref):
    h = lax.dot_general(x_ref[...], w_ref[...], (((1,), (1,)), ((), ())),
                        preferred_element_type=F32)
    dis = _dis_from_deg(deg_ref[...])
    z = jnp.zeros_like(h)
    h0_ref[...] = jnp.concatenate([h * dis[0][:, None], z], axis=1)
    h1_ref[...] = jnp.concatenate([h * dis[1][:, None], z], axis=1)
    h2_ref[...] = jnp.concatenate([h * dis[2][:, None], z], axis=1)


def _tc_mid_body(r0_ref, r1_ref, r2_ref, deg_ref, b_ref, w_ref,
                 o0_ref, o1_ref, o2_ref):
    dis = _dis_from_deg(deg_ref[...])
    ys = []
    for k, r_ref in enumerate((r0_ref, r1_ref, r2_ref)):
        r = r_ref[0, :, :F] + r_ref[1, :, :F]
        ys.append(jax.nn.relu(r * dis[k][:, None] + b_ref[...]))
    cat = jnp.concatenate(ys, axis=1)
    h = lax.dot_general(cat, w_ref[...], (((1,), (1,)), ((), ())),
                        preferred_element_type=F32)
    z = jnp.zeros_like(h)
    o0_ref[...] = jnp.concatenate([h * dis[0][:, None], z], axis=1)
    o1_ref[...] = jnp.concatenate([h * dis[1][:, None], z], axis=1)
    o2_ref[...] = jnp.concatenate([h * dis[2][:, None], z], axis=1)


def _tc_final_body(r0_ref, r1_ref, r2_ref, deg_ref, b_ref, w_ref, cb_ref,
                   out_ref):
    dis = _dis_from_deg(deg_ref[...])
    ys = []
    for k, r_ref in enumerate((r0_ref, r1_ref, r2_ref)):
        r = r_ref[0, :, :F] + r_ref[1, :, :F]
        ys.append(jax.nn.relu(r * dis[k][:, None] + b_ref[...]))
    cat = jnp.concatenate(ys, axis=1)
    out = lax.dot_general(cat, w_ref[...], (((1,), (1,)), ((), ())),
                          preferred_element_type=F32)
    out_ref[...] = out + cb_ref[...]


def _tc1(x_pad, lin1_w, degp, n_pad):
    grid = (n_pad // BN,)
    return pl.pallas_call(
        _tc1_body,
        grid=grid,
        in_specs=[
            pl.BlockSpec((BN, 128), lambda i: (i, 0)),
            pl.BlockSpec((F, 128), lambda i: (0, 0)),
            pl.BlockSpec((NC, 3, BN), lambda i: (0, 0, i)),
        ],
        out_specs=[pl.BlockSpec((BN, 2 * F), lambda i: (i, 0))] * 3,
        out_shape=[jax.ShapeDtypeStruct((n_pad, 2 * F), F32)] * 3,
    )(x_pad, lin1_w, degp)


def _tc_mid(r0, r1, r2, degp, bias, w, n_pad):
    grid = (n_pad // BN,)
    return pl.pallas_call(
        _tc_mid_body,
        grid=grid,
        in_specs=[
            pl.BlockSpec((NC, BN, 2 * F), lambda i: (0, i, 0)),
            pl.BlockSpec((NC, BN, 2 * F), lambda i: (0, i, 0)),
            pl.BlockSpec((NC, BN, 2 * F), lambda i: (0, i, 0)),
            pl.BlockSpec((NC, 3, BN), lambda i: (0, 0, i)),
            pl.BlockSpec((1, F), lambda i: (0, 0)),
            pl.BlockSpec((F, 3 * F), lambda i: (0, 0)),
        ],
        out_specs=[pl.BlockSpec((BN, 2 * F), lambda i: (i, 0))] * 3,
        out_shape=[jax.ShapeDtypeStruct((n_pad, 2 * F), F32)] * 3,
    )(r0, r1, r2, degp, bias, w)


def _tc_final(r0, r1, r2, degp, bias, w, cb, n_pad, d_out):
    grid = (n_pad // BN,)
    return pl.pallas_call(
        _tc_final_body,
        grid=grid,
        in_specs=[
            pl.BlockSpec((NC, BN, 2 * F), lambda i: (0, i, 0)),
            pl.BlockSpec((NC, BN, 2 * F), lambda i: (0, i, 0)),
            pl.BlockSpec((NC, BN, 2 * F), lambda i: (0, i, 0)),
            pl.BlockSpec((NC, 3, BN), lambda i: (0, 0, i)),
            pl.BlockSpec((1, F), lambda i: (0, 0)),
            pl.BlockSpec((d_out, 3 * F), lambda i: (0, 0)),
            pl.BlockSpec((1, d_out), lambda i: (0, 0)),
        ],
        out_specs=pl.BlockSpec((BN, d_out), lambda i: (i, 0)),
        out_shape=jax.ShapeDtypeStruct((n_pad, d_out), F32),
    )(r0, r1, r2, degp, bias, w, cb)


def _prep_edges(src, dst, w, e_pad, chunks, dummy):
    pad = e_pad - src.shape[0]
    src = jnp.concatenate([src.astype(jnp.int32),
                           jnp.full((pad,), dummy, jnp.int32)])
    dst = jnp.concatenate([dst.astype(jnp.int32),
                           jnp.full((pad,), dummy, jnp.int32)])
    w = jnp.concatenate([w.astype(F32), jnp.zeros((pad,), F32)])
    return (src.reshape(NW, chunks, CHUNK),
            dst.reshape(NW, chunks, CHUNK),
            w.reshape(NW, chunks, CHUNK))


def kernel(x, edge_index, edge_in, in_w, edge_out, out_w,
           lin1_w, bias1, linx0_w, biasx0, conv_w, conv_b):
    n, d_in = x.shape
    e = edge_index.shape[1]
    d_out = conv_w.shape[0]

    ept = _cdiv(e, NW * CHUNK) * CHUNK
    chunks = ept // CHUNK
    e_pad = ept * NW
    n_pad = _cdiv(n + 1, NS * CHUNK) * NS * CHUNK
    rows_per_tile = n_pad // NS

    ones = jnp.ones((e,), F32)
    s0, d0, w0 = _prep_edges(edge_index[0], edge_index[1], ones, e_pad, chunks, n)
    s1, d1, w1 = _prep_edges(edge_in[0], edge_in[1], in_w, e_pad, chunks, n)
    s2, d2, w2 = _prep_edges(edge_out[0], edge_out[1], out_w, e_pad, chunks, n)
    src_all = jnp.stack([s0, s1, s2])
    ew_all = jnp.stack([w0, w1, w2])

    x_pad = jnp.zeros((n_pad, d_in), F32).at[:n].set(x)
    zrow = jnp.zeros((rows_per_tile,), F32)
    zblk = jnp.zeros((CHUNK, 2 * F), F32)
    dummy_ew = jnp.zeros((8,), F32)

    def expand(w):
        return jnp.broadcast_to(w.reshape(NW, chunks, CHUNK, 1),
                                (NW, chunks, CHUNK, LANES)).reshape(-1)

    w1x = expand(w1)
    w2x = expand(w2)

    deg_kernel = _make_deg_kernel(n_pad, chunks)
    conv_u = _make_conv_kernel(n_pad, chunks, weighted=False)
    conv_w_kernel = _make_conv_kernel(n_pad, chunks, weighted=True)

    degp = deg_kernel(src_all, ew_all, zrow).reshape(NC, 3, n_pad)

    h0, h1, h2 = _tc1(x_pad, lin1_w, degp, n_pad)

    r0 = conv_u(h0, s0, d0, dummy_ew, zblk)
    r1 = conv_w_kernel(h1, s1, d1, w1x, zblk)
    r2 = conv_w_kernel(h2, s2, d2, w2x, zblk)

    g0, g1, g2 = _tc_mid(r0, r1, r2, degp, bias1, linx0_w, n_pad)

    q0 = conv_u(g0, s0, d0, dummy_ew, zblk)
    q1 = conv_w_kernel(g1, s1, d1, w1x, zblk)
    q2 = conv_w_kernel(g2, s2, d2, w2x, zblk)

    out = _tc_final(q0, q1, q2, degp, biasx0, conv_w, conv_b.reshape(1, d_out),
                    n_pad, d_out)
    return out[:n]
